# baseline (speedup 1.0000x reference)
import functools

import jax
import jax.numpy as jnp
from jax import lax
from jax.experimental import pallas as pl
from jax.experimental.pallas import tpu as pltpu
from jax.experimental.pallas import tpu_sc as plsc

VOCAB = 100000
EMBED_DIM = 64
BATCH = 16384

_NUM_CORES = 2
_NUM_SUBCORES = 16
_NUM_WORKERS = _NUM_CORES * _NUM_SUBCORES
_B_PER_W = BATCH // _NUM_WORKERS

_mesh = plsc.VectorSubcoreMesh(core_axis_name="c", subcore_axis_name="s")


@functools.partial(
    pl.kernel,
    mesh=_mesh,
    out_type=jax.ShapeDtypeStruct((BATCH, EMBED_DIM), jnp.float32),
    scratch_types=[
        pltpu.VMEM((_B_PER_W,), jnp.int32),
        pltpu.VMEM((_B_PER_W, EMBED_DIM), jnp.float32),
        pltpu.SemaphoreType.DMA,
    ],
    compiler_params=pltpu.CompilerParams(use_tc_tiling_on_sc=False),
)
def _gather_sc(idx_hbm, table_hbm, out_hbm, idx_v, rows_v, sem):
    wid = lax.axis_index("s") * _NUM_CORES + lax.axis_index("c")
    base = wid * _B_PER_W
    pltpu.sync_copy(idx_hbm.at[pl.ds(base, _B_PER_W)], idx_v)
    pltpu.async_copy(table_hbm.at[idx_v], rows_v, sem).wait()
    pltpu.sync_copy(rows_v, out_hbm.at[pl.ds(base, _B_PER_W)])


@jax.jit
def kernel(i, data):
    return _gather_sc(i, data)


# speedup vs baseline: 1.1543x; 1.1543x over previous
import functools

import jax
import jax.numpy as jnp
from jax import lax
from jax.experimental import pallas as pl
from jax.experimental.pallas import tpu as pltpu
from jax.experimental.pallas import tpu_sc as plsc

VOCAB = 100000
EMBED_DIM = 64
PAD_DIM = 128
BATCH = 16384

_NUM_CORES = 2
_NUM_SUBCORES = 16
_NUM_WORKERS = _NUM_CORES * _NUM_SUBCORES
_B_PER_W = BATCH // _NUM_WORKERS

_mesh = plsc.VectorSubcoreMesh(core_axis_name="c", subcore_axis_name="s")


@functools.partial(
    pl.kernel,
    mesh=_mesh,
    out_type=jax.ShapeDtypeStruct((BATCH, PAD_DIM), jnp.float32),
    scratch_types=[
        pltpu.VMEM((_B_PER_W,), jnp.int32),
        pltpu.VMEM((_B_PER_W, PAD_DIM), jnp.float32),
        pltpu.SemaphoreType.DMA,
    ],
)
def _gather_sc(idx_hbm, table_hbm, out_hbm, idx_v, rows_v, sem):
    wid = lax.axis_index("s") * _NUM_CORES + lax.axis_index("c")
    base = wid * _B_PER_W
    pltpu.sync_copy(idx_hbm.at[pl.ds(base, _B_PER_W)], idx_v)
    pltpu.async_copy(table_hbm.at[idx_v], rows_v, sem).wait()
    pltpu.sync_copy(rows_v, out_hbm.at[pl.ds(base, _B_PER_W)])


@jax.jit
def kernel(i, data):
    padded = jnp.pad(data, ((0, 0), (0, PAD_DIM - EMBED_DIM)))
    return _gather_sc(i, padded)[:, :EMBED_DIM]


# speedup vs baseline: 1.8327x; 1.5878x over previous
import functools

import jax
import jax.numpy as jnp
from jax import lax
from jax.experimental import pallas as pl
from jax.experimental.pallas import tpu as pltpu
from jax.experimental.pallas import tpu_sc as plsc

VOCAB = 100000
EMBED_DIM = 64
BATCH = 16384

_NUM_CORES = 2
_NUM_SUBCORES = 16
_NUM_WORKERS = _NUM_CORES * _NUM_SUBCORES
_DIMS_PER_W = EMBED_DIM // _NUM_WORKERS
_OUT_CHUNK = 4096

_mesh = plsc.VectorSubcoreMesh(core_axis_name="c", subcore_axis_name="s")


@functools.partial(
    pl.kernel,
    mesh=_mesh,
    out_type=jax.ShapeDtypeStruct((EMBED_DIM, BATCH), jnp.float32),
    scratch_types=[
        pltpu.VMEM((BATCH,), jnp.int32),
        pltpu.VMEM((VOCAB,), jnp.float32),
        pltpu.VMEM((_OUT_CHUNK,), jnp.float32),
        pltpu.SemaphoreType.DMA,
    ],
    compiler_params=pltpu.CompilerParams(needs_layout_passes=False),
)
def _gather_sc(idx_hbm, tablet_hbm, outt_hbm, idx_v, row_v, out_v, sem):
    wid = lax.axis_index("s") * _NUM_CORES + lax.axis_index("c")
    pltpu.sync_copy(idx_hbm, idx_v)
    for r in range(_DIMS_PER_W):
        c = wid * _DIMS_PER_W + r
        pltpu.sync_copy(tablet_hbm.at[c], row_v)

        def chunk_body(ck, carry):
            def grp_body(g, carry2):
                idx16 = idx_v[pl.ds(ck * _OUT_CHUNK + g * 16, 16)]
                out_v[pl.ds(g * 16, 16)] = plsc.load_gather(row_v, [idx16])
                return carry2

            lax.fori_loop(0, _OUT_CHUNK // 16, grp_body, 0)
            pltpu.sync_copy(out_v,
                            outt_hbm.at[c, pl.ds(ck * _OUT_CHUNK, _OUT_CHUNK)])
            return carry

        lax.fori_loop(0, BATCH // _OUT_CHUNK, chunk_body, 0)


@jax.jit
def kernel(i, data):
    return _gather_sc(i, data.T).T


# speedup vs baseline: 1.9370x; 1.0569x over previous
import functools

import jax
import jax.numpy as jnp
from jax import lax
from jax.experimental import pallas as pl
from jax.experimental.pallas import tpu as pltpu
from jax.experimental.pallas import tpu_sc as plsc

VOCAB = 100000
EMBED_DIM = 64
BATCH = 16384

_NUM_CORES = 2
_NUM_SUBCORES = 16
_NUM_WORKERS = _NUM_CORES * _NUM_SUBCORES
_DIMS_PER_W = EMBED_DIM // _NUM_WORKERS
_OUT_CHUNK = 4096
_NCHUNK = BATCH // _OUT_CHUNK

_mesh = plsc.VectorSubcoreMesh(core_axis_name="c", subcore_axis_name="s")


@functools.partial(
    pl.kernel,
    mesh=_mesh,
    out_type=jax.ShapeDtypeStruct((EMBED_DIM, BATCH), jnp.float32),
    scratch_types=[
        pltpu.VMEM((BATCH,), jnp.int32),
        pltpu.VMEM((VOCAB,), jnp.float32),
        pltpu.VMEM((_OUT_CHUNK,), jnp.float32),
        pltpu.VMEM((_OUT_CHUNK,), jnp.float32),
        pltpu.SemaphoreType.DMA,
        pltpu.SemaphoreType.DMA,
        pltpu.SemaphoreType.DMA,
    ],
    compiler_params=pltpu.CompilerParams(needs_layout_passes=False),
)
def _gather_sc(idx_hbm, tablet_hbm, outt_hbm, idx_v, row_v, out_a, out_b,
               sem_row, sem_a, sem_b):
    wid = lax.axis_index("s") * _NUM_CORES + lax.axis_index("c")
    row_copy = pltpu.async_copy(tablet_hbm.at[wid * _DIMS_PER_W], row_v,
                                sem_row)
    pltpu.sync_copy(idx_hbm, idx_v)
    row_copy.wait()

    out_bufs = (out_a, out_b)
    out_sems = (sem_a, sem_b)

    for r in range(_DIMS_PER_W):
        c = wid * _DIMS_PER_W + r
        for ck in range(_NCHUNK):
            buf = out_bufs[ck % 2]
            if r > 0 or ck >= 2:
                pltpu.make_async_copy(
                    buf, outt_hbm.at[c, pl.ds(0, _OUT_CHUNK)],
                    out_sems[ck % 2]).wait()

            def grp_body(g, carry):
                idx16 = idx_v[pl.ds(ck * _OUT_CHUNK + g * 16, 16)]
                buf[pl.ds(g * 16, 16)] = plsc.load_gather(row_v, [idx16])
                return carry

            lax.fori_loop(0, _OUT_CHUNK // 16, grp_body, 0, unroll=8)
            pltpu.async_copy(buf,
                             outt_hbm.at[c, pl.ds(ck * _OUT_CHUNK, _OUT_CHUNK)],
                             out_sems[ck % 2])
        if r + 1 < _DIMS_PER_W:
            pltpu.sync_copy(tablet_hbm.at[c + 1], row_v)

    for k in range(2):
        pltpu.make_async_copy(
            out_bufs[k],
            outt_hbm.at[wid * _DIMS_PER_W + _DIMS_PER_W - 1,
                        pl.ds(0, _OUT_CHUNK)],
            out_sems[k]).wait()


@jax.jit
def kernel(i, data):
    return _gather_sc(i, data.T).T
